# Initial kernel scaffold; baseline (speedup 1.0000x reference)
import jax
import jax.numpy as jnp
from jax.experimental import pallas as pl


def kernel(x1, edge_slices, W1, b1, W2, b2, W3, b3):
    raise NotImplementedError("write your pallas kernel here")


# speedup vs baseline: 23.4861x; 23.4861x over previous
import functools

import jax
import jax.numpy as jnp
from jax.experimental import pallas as pl
from jax.experimental.pallas import tpu as pltpu

_ROWS_PER_BLOCK = 2560


def _fused_body(lo_ref, hi_ref, x_ref, w1_ref, b1_ref, w2_ref, b2_ref,
                w3_ref, b3_ref, out_ref, acc_ref, *, num_blocks, rows):
    b = pl.program_id(0)
    a = jnp.dot(x_ref[...], w1_ref[...], preferred_element_type=jnp.float32)
    a = jnp.maximum(a + b1_ref[...], 0.0)

    gid = b * rows + jax.lax.broadcasted_iota(jnp.int32, (1, rows), 1)
    onehot = ((lo_ref[...] <= gid) & (gid < hi_ref[...])).astype(jnp.float32)
    partial = jnp.dot(onehot, a, preferred_element_type=jnp.float32)

    @pl.when(b == 0)
    def _init():
        acc_ref[...] = partial

    @pl.when(b > 0)
    def _accum():
        acc_ref[...] += partial

    @pl.when(b == num_blocks - 1)
    def _finalize():
        counts = jnp.maximum((hi_ref[...] - lo_ref[...]).astype(jnp.float32), 1.0)
        mean = acc_ref[...] / counts
        h2 = jnp.dot(mean, w2_ref[...], preferred_element_type=jnp.float32) + b2_ref[...]
        out_ref[...] = jnp.dot(h2, w3_ref[...], preferred_element_type=jnp.float32) + b3_ref[...]


def kernel(x1, edge_slices, W1, b1, W2, b2, W3, b3):
    n, d_in = x1.shape
    d_out = W2.shape[1]
    s = edge_slices.shape[0] - 1
    rows = _ROWS_PER_BLOCK
    num_blocks = n // rows
    assert num_blocks * rows == n

    lo = edge_slices[:-1].reshape(s, 1)
    hi = edge_slices[1:].reshape(s, 1)

    body = functools.partial(_fused_body, num_blocks=num_blocks, rows=rows)
    full = lambda shape: pl.BlockSpec(shape, lambda b: (0, 0))
    out = pl.pallas_call(
        body,
        grid=(num_blocks,),
        in_specs=[
            full((s, 1)),
            full((s, 1)),
            pl.BlockSpec((rows, d_in), lambda b: (b, 0)),
            full((d_in, d_out)),
            full((1, d_out)),
            full((d_out, d_out)),
            full((1, d_out)),
            full((d_out, d_out)),
            full((1, d_out)),
        ],
        out_specs=full((s, d_out)),
        out_shape=jax.ShapeDtypeStruct((s, d_out), jnp.float32),
        scratch_shapes=[pltpu.VMEM((s, d_out), jnp.float32)],
        compiler_params=pltpu.CompilerParams(
            dimension_semantics=("arbitrary",),
        ),
    )(lo, hi, x1, W1, b1.reshape(1, d_out), W2, b2.reshape(1, d_out),
      W3, b3.reshape(1, d_out))
    return out
